# baseline (speedup 1.0000x reference)
import functools

import jax
import jax.numpy as jnp
from jax import lax
from jax.experimental import pallas as pl
from jax.experimental.pallas import tpu as pltpu
from jax.experimental.pallas import tpu_sc as plsc

N = 10000
D = 128
G = 256
L = 3
NP = 10240
NGROUPS = NP // 128


def _mesh():
    return plsc.VectorSubcoreMesh(core_axis_name="c", subcore_axis_name="s")


def _segsum_body(gpw, emb_mode, h_hbm, src_hbm, dst_hbm, agg_hbm,
                 src_v, dst_v, r0_v, r1_v, zed_v, h_sp, agg_sp,
                 g0_s, g1_s, idx_hbm=None, idx_v=None):
    rows = (r0_v, r1_v)
    gsem = (g0_s, g1_s)
    c = lax.axis_index("c")
    s = lax.axis_index("s")
    col0 = c * 64
    row0 = s * (NP // 16)

    if emb_mode:
        pltpu.sync_copy(idx_hbm.at[pl.ds(row0, NP // 16)], idx_v)
        for k in range((NP // 16) // 128):
            pltpu.async_copy(h_hbm.at[c].at[idx_v.at[pl.ds(k * 128, 128)]],
                             r0_v.at[pl.ds(0, 128), :], g0_s).wait()
            pltpu.sync_copy(r0_v.at[pl.ds(0, 128), :],
                            h_sp.at[pl.ds(row0 + k * 128, 128), :])
    else:
        pltpu.sync_copy(h_hbm.at[pl.ds(row0, NP // 16), pl.ds(col0, 64)],
                        h_sp.at[pl.ds(row0, NP // 16), :])

    def zstore(i, carry):
        zed_v[i // 4, pl.ds((i % 4) * 16, 16)] = jnp.zeros((16,), jnp.float32)
        return carry

    lax.fori_loop(0, 64 * 4, zstore, 0)

    def zcp(i, carry):
        pltpu.sync_copy(zed_v, agg_sp.at[pl.ds(row0 + i * 64, 64), :])
        return carry

    lax.fori_loop(0, (NP // 16) // 64, zcp, 0)

    plsc.subcore_barrier()

    g0 = s * gpw

    def chunk(ci, carry):
        cg = g0 + ci * 16
        pltpu.sync_copy(src_hbm.at[pl.ds(cg, 16), :], src_v)
        pltpu.sync_copy(dst_hbm.at[pl.ds(cg, 16), :], dst_v)

        pltpu.async_copy(h_sp.at[src_v.at[0]], rows[0], gsem[0])

        def pair(jj, carry2):
            j0 = jj * 2
            j1 = j0 + 1
            pltpu.async_copy(h_sp.at[src_v.at[j1]], rows[1], gsem[1])
            pltpu.make_async_copy(h_sp.at[src_v.at[j0]], rows[0],
                                  gsem[0]).wait()
            pltpu.sync_copy(rows[0], agg_sp.at[dst_v.at[j0]], add=True)

            @pl.when(jj < 7)
            def _():
                pltpu.async_copy(h_sp.at[src_v.at[j0 + 2]], rows[0], gsem[0])

            pltpu.make_async_copy(h_sp.at[src_v.at[j1]], rows[1],
                                  gsem[1]).wait()
            pltpu.sync_copy(rows[1], agg_sp.at[dst_v.at[j1]], add=True)
            return carry2

        lax.fori_loop(0, 8, pair, 0)
        return carry

    lax.fori_loop(0, gpw // 16, chunk, 0)

    plsc.subcore_barrier()
    pltpu.sync_copy(agg_sp.at[pl.ds(row0, NP // 16), :],
                    agg_hbm.at[pl.ds(row0, NP // 16), pl.ds(col0, 64)])


@functools.lru_cache(maxsize=None)
def _segsum_call(gpw, emb_mode):
    scratch = [
        pltpu.VMEM((16, 256), jnp.int32),
        pltpu.VMEM((16, 256), jnp.int32),
        pltpu.VMEM((256, 64), jnp.float32),
        pltpu.VMEM((256, 64), jnp.float32),
        pltpu.VMEM((64, 64), jnp.float32),
        pltpu.VMEM_SHARED((NP, 64), jnp.float32),
        pltpu.VMEM_SHARED((NP, 64), jnp.float32),
        pltpu.SemaphoreType.DMA,
        pltpu.SemaphoreType.DMA,
    ]
    if emb_mode:
        scratch = scratch + [pltpu.VMEM((NP // 16,), jnp.int32)]

        def body(emb2_hbm, src_hbm, dst_hbm, idx_hbm, agg_hbm,
                 src_v, dst_v, r0_v, r1_v, zed_v, h_sp, agg_sp,
                 g0_s, g1_s, idx_v):
            _segsum_body(gpw, True, emb2_hbm, src_hbm, dst_hbm, agg_hbm,
                         src_v, dst_v, r0_v, r1_v, zed_v, h_sp, agg_sp,
                         g0_s, g1_s, idx_hbm=idx_hbm, idx_v=idx_v)
    else:
        def body(h_hbm, src_hbm, dst_hbm, agg_hbm,
                 src_v, dst_v, r0_v, r1_v, zed_v, h_sp, agg_sp,
                 g0_s, g1_s):
            _segsum_body(gpw, False, h_hbm, src_hbm, dst_hbm, agg_hbm,
                         src_v, dst_v, r0_v, r1_v, zed_v, h_sp, agg_sp,
                         g0_s, g1_s)

    return pl.kernel(
        body,
        out_type=jax.ShapeDtypeStruct((NP, D), jnp.float32),
        mesh=_mesh(),
        compiler_params=pltpu.CompilerParams(use_tc_tiling_on_sc=False),
        scratch_types=scratch,
    )


def _mlp_body(emb_mode, h_ref, agg_ref, w1_ref, b1_ref, w2_ref, b2_ref,
              ga_ref, be_ref, o_ref):
    if emb_mode:
        idx_ref, emb_ref = h_ref
        oh = (idx_ref[...] ==
              lax.broadcasted_iota(jnp.int32, (NP, 128), 1)
              ).astype(jnp.float32)
        h = jnp.dot(oh, emb_ref[...], preferred_element_type=jnp.float32)
    else:
        h = h_ref[...]
    z = h + agg_ref[...]
    z = jnp.maximum(
        jnp.dot(z, w1_ref[...], preferred_element_type=jnp.float32)
        + b1_ref[...], 0.0)
    z = jnp.maximum(
        jnp.dot(z, w2_ref[...], preferred_element_type=jnp.float32)
        + b2_ref[...], 0.0)
    msk = (lax.broadcasted_iota(jnp.int32, (NP, 1), 0) < N).astype(jnp.float32)
    zm = z * msk
    mean = jnp.sum(zm, axis=0, keepdims=True) * (1.0 / N)
    diff = (z - mean) * msk
    var = jnp.sum(diff * diff, axis=0, keepdims=True) * (1.0 / N)
    o_ref[...] = ((z - mean) * lax.rsqrt(var + 1e-5) * ga_ref[...]
                  + be_ref[...])


def _mlp(h, agg, w1, b1, w2, b2, ga, be):
    emb_mode = isinstance(h, tuple)
    return pl.pallas_call(
        functools.partial(_mlp_body, emb_mode),
        out_shape=jax.ShapeDtypeStruct((NP, D), jnp.float32),
    )(h, agg, w1, b1, w2, b2, ga, be)


def _pool_body(z1_hbm, z2_hbm, z3_hbm, bt_hbm, out_hbm,
               bt_v, acc_v, bA1, bA2, bA3, bB1, bB2, bB3,
               sA1, sA2, sA3, sB1, sB2, sB3):
    set_a = ((bA1, sA1, z1_hbm), (bA2, sA2, z2_hbm), (bA3, sA3, z3_hbm))
    set_b = ((bB1, sB1, z1_hbm), (bB2, sB2, z2_hbm), (bB3, sB3, z3_hbm))
    c = lax.axis_index("c")
    s = lax.axis_index("s")
    t = s * 2 + c
    glo = t * 8
    ghi = glo + 8

    pltpu.sync_copy(bt_hbm, bt_v.at[pl.ds(0, NP)])
    bt_v[pl.ds(NP, 16)] = jnp.full((16,), 2 ** 30, jnp.int32)

    def lower_bound(val):
        def body(i, st):
            lo, hi = st
            mid = (lo + hi) // 2
            v = bt_v[pl.ds(mid, 16)][0]
            pred = v < val
            return (jnp.where(pred, mid + 1, lo), jnp.where(pred, hi, mid))

        return lax.fori_loop(0, 14, body, (jnp.int32(0), jnp.int32(NP)))[0]

    s0 = lower_bound(glo)
    e0 = lower_bound(ghi)

    neg = jnp.full((16,), -jnp.inf, dtype=jnp.float32)

    def ini(i, carry):
        acc_v[i // 24, pl.ds((i % 24) * 16, 16)] = neg
        return carry

    lax.fori_loop(0, 8 * 24, ini, 0)

    def issue(st, bi):
        r0 = pl.multiple_of(bi * 32, 32)
        for buf, sem, z in st:
            pltpu.async_copy(z.at[pl.ds(r0, 32), :], buf, sem)

    def wait(st, bi):
        r0 = pl.multiple_of(bi * 32, 32)
        for buf, sem, z in st:
            pltpu.make_async_copy(z.at[pl.ds(r0, 32), :], buf, sem).wait()

    def compute(st, bi):
        r0 = pl.multiple_of(bi * 32, 32)
        bufs = [x[0] for x in st]
        bv_lo = bt_v[pl.ds(r0, 16)]
        bv_hi = bt_v[pl.ds(r0 + 16, 16)]
        for k in range(32):
            r = r0 + k
            bv = bv_lo if k < 16 else bv_hi

            @pl.when((r >= s0) & (r < e0))
            def _(k=k, bv=bv):
                g = bv[k % 16] - glo
                for bidx, buf in enumerate(bufs):
                    def feat(j, c3, buf=buf, base=bidx * 128, k=k, g=g):
                        off = base + j * 16
                        acc_v[g, pl.ds(off, 16)] = jnp.maximum(
                            acc_v[g, pl.ds(off, 16)],
                            buf[k, pl.ds(j * 16, 16)])
                        return c3

                    lax.fori_loop(0, 8, feat, 0)

    b0 = s0 // 32
    bend = (e0 + 31) // 32

    @pl.when(bend > b0)
    def _():
        issue(set_a, b0)

    def blk2(jj, carry):
        i0 = b0 + 2 * jj
        i1 = i0 + 1

        @pl.when(i1 < bend)
        def _():
            issue(set_b, i1)

        wait(set_a, i0)
        compute(set_a, i0)

        @pl.when(i1 + 1 < bend)
        def _():
            issue(set_a, i0 + 2)

        @pl.when(i1 < bend)
        def _():
            wait(set_b, i1)
            compute(set_b, i1)

        return carry

    lax.fori_loop(0, (bend - b0 + 1) // 2, blk2, 0)

    pltpu.sync_copy(acc_v, out_hbm.at[pl.ds(glo, 8), :])


@functools.lru_cache(maxsize=None)
def _pool_call():
    return pl.kernel(
        _pool_body,
        out_type=jax.ShapeDtypeStruct((G, L * D), jnp.float32),
        mesh=_mesh(),
        scratch_types=[
            pltpu.VMEM((NP + 16,), jnp.int32),
            pltpu.VMEM((8, L * D), jnp.float32),
            pltpu.VMEM((32, D), jnp.float32),
            pltpu.VMEM((32, D), jnp.float32),
            pltpu.VMEM((32, D), jnp.float32),
            pltpu.VMEM((32, D), jnp.float32),
            pltpu.VMEM((32, D), jnp.float32),
            pltpu.VMEM((32, D), jnp.float32),
            pltpu.SemaphoreType.DMA,
            pltpu.SemaphoreType.DMA,
            pltpu.SemaphoreType.DMA,
            pltpu.SemaphoreType.DMA,
            pltpu.SemaphoreType.DMA,
            pltpu.SemaphoreType.DMA,
        ],
    )


def kernel(x, edge_index, batch, emb, W1, b1, W2, b2, gamma, beta):
    idx = x[:, 0].astype(jnp.int32)
    n = idx.shape[0]
    idx_p = jnp.pad(idx, (0, NP - n))
    bt_p = jnp.pad(batch.astype(jnp.int32), (0, NP - n), constant_values=G)

    v = emb.shape[0]
    emb_pad = jnp.pad(emb, ((0, 128 - v), (0, 0)))
    emb2 = jnp.stack([emb[:, :64], emb[:, 64:]])

    src = edge_index[0].astype(jnp.int32)
    dst = edge_index[1].astype(jnp.int32)
    e = src.shape[0]
    egroups = -(-e // 256)
    gpw = -(-egroups // 16)
    gpw = -(-gpw // 16) * 16
    ep = gpw * 16 * 256
    src_p = jnp.pad(src, (0, ep - e)).reshape(gpw * 16, 256)
    dst_p = jnp.pad(dst, (0, ep - e), constant_values=NP - 1).reshape(
        gpw * 16, 256)

    outs = []
    h = None
    for i in range(L):
        if i == 0:
            agg = _segsum_call(gpw, True)(emb2, src_p, dst_p, idx_p)
            hin = (idx_p.reshape(NP, 1), emb_pad)
        else:
            agg = _segsum_call(gpw, False)(h, src_p, dst_p)
            hin = h
        h = _mlp(hin, agg, W1[i], b1[i].reshape(1, D),
                 W2[i], b2[i].reshape(1, D),
                 gamma[i].reshape(1, D), beta[i].reshape(1, D))
        outs.append(h)

    return _pool_call()(outs[0], outs[1], outs[2], bt_p)


# speedup vs baseline: 1.0028x; 1.0028x over previous
import functools

import jax
import jax.numpy as jnp
from jax import lax
from jax.experimental import pallas as pl
from jax.experimental.pallas import tpu as pltpu
from jax.experimental.pallas import tpu_sc as plsc

N = 10000
D = 128
G = 256
L = 3
NP = 10240
NGROUPS = NP // 128


def _mesh():
    return plsc.VectorSubcoreMesh(core_axis_name="c", subcore_axis_name="s")


def _emb_body(idx_hbm, emb_hbm, h_hbm, idx_v, rows_v, sem):
    c = lax.axis_index("c")
    s = lax.axis_index("s")
    w = s * 2 + c

    def do(j, carry):
        gid = w + 32 * j

        @pl.when(gid < NGROUPS)
        def _():
            pltpu.sync_copy(idx_hbm.at[gid], idx_v)
            pltpu.async_copy(emb_hbm.at[idx_v], rows_v, sem).wait()
            pltpu.sync_copy(rows_v, h_hbm.at[pl.ds(gid * 128, 128), :])

        return carry

    lax.fori_loop(0, (NGROUPS + 31) // 32, do, 0)


@functools.lru_cache(maxsize=None)
def _emb_call():
    return pl.kernel(
        _emb_body,
        out_type=jax.ShapeDtypeStruct((NP, D), jnp.float32),
        mesh=_mesh(),
        scratch_types=[
            pltpu.VMEM((128,), jnp.int32),
            pltpu.VMEM((128, D), jnp.float32),
            pltpu.SemaphoreType.DMA,
        ],
    )


def _segsum_body(gpw, h_hbm, src_hbm, dst_hbm, agg_hbm,
                 src_v, dst_v, r0_v, r1_v, zed_v, h_sp, agg_sp,
                 g0_s, g1_s):
    rows = (r0_v, r1_v)
    gsem = (g0_s, g1_s)
    c = lax.axis_index("c")
    s = lax.axis_index("s")
    col0 = c * 64
    row0 = s * (NP // 16)

    pltpu.sync_copy(h_hbm.at[pl.ds(row0, NP // 16), pl.ds(col0, 64)],
                    h_sp.at[pl.ds(row0, NP // 16), :])

    def zstore(i, carry):
        zed_v[i // 4, pl.ds((i % 4) * 16, 16)] = jnp.zeros((16,), jnp.float32)
        return carry

    lax.fori_loop(0, 64 * 4, zstore, 0)

    def zcp(i, carry):
        pltpu.sync_copy(zed_v, agg_sp.at[pl.ds(row0 + i * 64, 64), :])
        return carry

    lax.fori_loop(0, (NP // 16) // 64, zcp, 0)

    plsc.subcore_barrier()

    g0 = s * gpw

    def chunk(ci, carry):
        cg = g0 + ci * 16
        pltpu.sync_copy(src_hbm.at[pl.ds(cg, 16), :], src_v)
        pltpu.sync_copy(dst_hbm.at[pl.ds(cg, 16), :], dst_v)

        pltpu.async_copy(h_sp.at[src_v.at[0]], rows[0], gsem[0])

        def pair(jj, carry2):
            j0 = jj * 2
            j1 = j0 + 1
            pltpu.async_copy(h_sp.at[src_v.at[j1]], rows[1], gsem[1])
            pltpu.make_async_copy(h_sp.at[src_v.at[j0]], rows[0],
                                  gsem[0]).wait()
            pltpu.sync_copy(rows[0], agg_sp.at[dst_v.at[j0]], add=True)

            @pl.when(jj < 7)
            def _():
                pltpu.async_copy(h_sp.at[src_v.at[j0 + 2]], rows[0], gsem[0])

            pltpu.make_async_copy(h_sp.at[src_v.at[j1]], rows[1],
                                  gsem[1]).wait()
            pltpu.sync_copy(rows[1], agg_sp.at[dst_v.at[j1]], add=True)
            return carry2

        lax.fori_loop(0, 8, pair, 0)
        return carry

    lax.fori_loop(0, gpw // 16, chunk, 0)

    plsc.subcore_barrier()
    pltpu.sync_copy(agg_sp.at[pl.ds(row0, NP // 16), :],
                    agg_hbm.at[pl.ds(row0, NP // 16), pl.ds(col0, 64)])


@functools.lru_cache(maxsize=None)
def _segsum_call(gpw):
    return pl.kernel(
        functools.partial(_segsum_body, gpw),
        out_type=jax.ShapeDtypeStruct((NP, D), jnp.float32),
        mesh=_mesh(),
        compiler_params=pltpu.CompilerParams(use_tc_tiling_on_sc=False),
        scratch_types=[
            pltpu.VMEM((16, 256), jnp.int32),
            pltpu.VMEM((16, 256), jnp.int32),
            pltpu.VMEM((256, 64), jnp.float32),
            pltpu.VMEM((256, 64), jnp.float32),
            pltpu.VMEM((64, 64), jnp.float32),
            pltpu.VMEM_SHARED((NP, 64), jnp.float32),
            pltpu.VMEM_SHARED((NP, 64), jnp.float32),
            pltpu.SemaphoreType.DMA,
            pltpu.SemaphoreType.DMA,
        ],
    )


def _mlp_body(h_ref, agg_ref, w1_ref, b1_ref, w2_ref, b2_ref,
              ga_ref, be_ref, o_ref):
    z = h_ref[...] + agg_ref[...]
    z = jnp.maximum(
        jnp.dot(z, w1_ref[...], preferred_element_type=jnp.float32)
        + b1_ref[...], 0.0)
    z = jnp.maximum(
        jnp.dot(z, w2_ref[...], preferred_element_type=jnp.float32)
        + b2_ref[...], 0.0)
    msk = (lax.broadcasted_iota(jnp.int32, (NP, 1), 0) < N).astype(jnp.float32)
    zm = z * msk
    mean = jnp.sum(zm, axis=0, keepdims=True) * (1.0 / N)
    diff = (z - mean) * msk
    var = jnp.sum(diff * diff, axis=0, keepdims=True) * (1.0 / N)
    o_ref[...] = ((z - mean) * lax.rsqrt(var + 1e-5) * ga_ref[...]
                  + be_ref[...])


def _mlp(h, agg, w1, b1, w2, b2, ga, be):
    return pl.pallas_call(
        _mlp_body,
        out_shape=jax.ShapeDtypeStruct((NP, D), jnp.float32),
    )(h, agg, w1, b1, w2, b2, ga, be)


def _pool_body(z1_hbm, z2_hbm, z3_hbm, bt_hbm, out_hbm,
               bt_v, acc_v, bA1, bA2, bA3, bB1, bB2, bB3,
               sA1, sA2, sA3, sB1, sB2, sB3):
    set_a = ((bA1, sA1, z1_hbm), (bA2, sA2, z2_hbm), (bA3, sA3, z3_hbm))
    set_b = ((bB1, sB1, z1_hbm), (bB2, sB2, z2_hbm), (bB3, sB3, z3_hbm))
    c = lax.axis_index("c")
    s = lax.axis_index("s")
    t = s * 2 + c
    glo = t * 8
    ghi = glo + 8

    pltpu.sync_copy(bt_hbm, bt_v.at[pl.ds(0, NP)])
    bt_v[pl.ds(NP, 16)] = jnp.full((16,), 2 ** 30, jnp.int32)

    def lower_bound(val):
        def body(i, st):
            lo, hi = st
            mid = (lo + hi) // 2
            v = bt_v[pl.ds(mid, 16)][0]
            pred = v < val
            return (jnp.where(pred, mid + 1, lo), jnp.where(pred, hi, mid))

        return lax.fori_loop(0, 14, body, (jnp.int32(0), jnp.int32(NP)))[0]

    s0 = lower_bound(glo)
    e0 = lower_bound(ghi)

    neg = jnp.full((16,), -jnp.inf, dtype=jnp.float32)

    def ini(i, carry):
        acc_v[i // 24, pl.ds((i % 24) * 16, 16)] = neg
        return carry

    lax.fori_loop(0, 8 * 24, ini, 0)

    def issue(st, bi):
        r0 = pl.multiple_of(bi * 32, 32)
        for buf, sem, z in st:
            pltpu.async_copy(z.at[pl.ds(r0, 32), :], buf, sem)

    def wait(st, bi):
        r0 = pl.multiple_of(bi * 32, 32)
        for buf, sem, z in st:
            pltpu.make_async_copy(z.at[pl.ds(r0, 32), :], buf, sem).wait()

    def compute(st, bi):
        r0 = pl.multiple_of(bi * 32, 32)
        bufs = [x[0] for x in st]
        bv_lo = bt_v[pl.ds(r0, 16)]
        bv_hi = bt_v[pl.ds(r0 + 16, 16)]
        for k in range(32):
            r = r0 + k
            bv = bv_lo if k < 16 else bv_hi

            @pl.when((r >= s0) & (r < e0))
            def _(k=k, bv=bv):
                g = bv[k % 16] - glo
                for bidx, buf in enumerate(bufs):
                    def feat(j, c3, buf=buf, base=bidx * 128, k=k, g=g):
                        off = base + j * 16
                        acc_v[g, pl.ds(off, 16)] = jnp.maximum(
                            acc_v[g, pl.ds(off, 16)],
                            buf[k, pl.ds(j * 16, 16)])
                        return c3

                    lax.fori_loop(0, 8, feat, 0)

    b0 = s0 // 32
    bend = (e0 + 31) // 32

    @pl.when(bend > b0)
    def _():
        issue(set_a, b0)

    def blk2(jj, carry):
        i0 = b0 + 2 * jj
        i1 = i0 + 1

        @pl.when(i1 < bend)
        def _():
            issue(set_b, i1)

        wait(set_a, i0)
        compute(set_a, i0)

        @pl.when(i1 + 1 < bend)
        def _():
            issue(set_a, i0 + 2)

        @pl.when(i1 < bend)
        def _():
            wait(set_b, i1)
            compute(set_b, i1)

        return carry

    lax.fori_loop(0, (bend - b0 + 1) // 2, blk2, 0)

    pltpu.sync_copy(acc_v, out_hbm.at[pl.ds(glo, 8), :])


@functools.lru_cache(maxsize=None)
def _pool_call():
    return pl.kernel(
        _pool_body,
        out_type=jax.ShapeDtypeStruct((G, L * D), jnp.float32),
        mesh=_mesh(),
        scratch_types=[
            pltpu.VMEM((NP + 16,), jnp.int32),
            pltpu.VMEM((8, L * D), jnp.float32),
            pltpu.VMEM((32, D), jnp.float32),
            pltpu.VMEM((32, D), jnp.float32),
            pltpu.VMEM((32, D), jnp.float32),
            pltpu.VMEM((32, D), jnp.float32),
            pltpu.VMEM((32, D), jnp.float32),
            pltpu.VMEM((32, D), jnp.float32),
            pltpu.SemaphoreType.DMA,
            pltpu.SemaphoreType.DMA,
            pltpu.SemaphoreType.DMA,
            pltpu.SemaphoreType.DMA,
            pltpu.SemaphoreType.DMA,
            pltpu.SemaphoreType.DMA,
        ],
    )


def kernel(x, edge_index, batch, emb, W1, b1, W2, b2, gamma, beta):
    idx = x[:, 0].astype(jnp.int32)
    n = idx.shape[0]
    idx_p = jnp.pad(idx, (0, NP - n))
    bt_p = jnp.pad(batch.astype(jnp.int32), (0, NP - n), constant_values=G)

    src = edge_index[0].astype(jnp.int32)
    dst = edge_index[1].astype(jnp.int32)
    e = src.shape[0]
    egroups = -(-e // 256)
    gpw = -(-egroups // 16)
    gpw = -(-gpw // 16) * 16
    ep = gpw * 16 * 256
    src_p = jnp.pad(src, (0, ep - e)).reshape(gpw * 16, 256)
    dst_p = jnp.pad(dst, (0, ep - e), constant_values=NP - 1).reshape(
        gpw * 16, 256)

    h = _emb_call()(idx_p.reshape(NGROUPS, 128), emb)

    outs = []
    for i in range(L):
        agg = _segsum_call(gpw)(h, src_p, dst_p)
        h = _mlp(h, agg, W1[i], b1[i].reshape(1, D),
                 W2[i], b2[i].reshape(1, D),
                 gamma[i].reshape(1, D), beta[i].reshape(1, D))
        outs.append(h)

    return _pool_call()(outs[0], outs[1], outs[2], bt_p)


# speedup vs baseline: 1.0330x; 1.0301x over previous
import functools

import jax
import jax.numpy as jnp
from jax import lax
from jax.experimental import pallas as pl
from jax.experimental.pallas import tpu as pltpu
from jax.experimental.pallas import tpu_sc as plsc

N = 10000
D = 128
G = 256
L = 3
NP = 10240
NGROUPS = NP // 128


def _mesh():
    return plsc.VectorSubcoreMesh(core_axis_name="c", subcore_axis_name="s")


def _emb_body(idx_hbm, emb_hbm, h_hbm, idx_v, rows_v, sem):
    c = lax.axis_index("c")
    s = lax.axis_index("s")
    w = s * 2 + c

    def do(j, carry):
        gid = w + 32 * j

        @pl.when(gid < NGROUPS)
        def _():
            pltpu.sync_copy(idx_hbm.at[gid], idx_v)
            pltpu.async_copy(emb_hbm.at[idx_v], rows_v, sem).wait()
            pltpu.sync_copy(rows_v, h_hbm.at[pl.ds(gid * 128, 128), :])

        return carry

    lax.fori_loop(0, (NGROUPS + 31) // 32, do, 0)


@functools.lru_cache(maxsize=None)
def _emb_call():
    return pl.kernel(
        _emb_body,
        out_type=jax.ShapeDtypeStruct((NP, D), jnp.float32),
        mesh=_mesh(),
        scratch_types=[
            pltpu.VMEM((128,), jnp.int32),
            pltpu.VMEM((128, D), jnp.float32),
            pltpu.SemaphoreType.DMA,
        ],
    )


def _segsum_body(gpw, h_hbm, sd_hbm, agg_hbm,
                 sd_v, r0_v, r1_v, zed_v, h_sp, agg_sp,
                 g0_s, g1_s):
    rows = (r0_v, r1_v)
    gsem = (g0_s, g1_s)
    c = lax.axis_index("c")
    s = lax.axis_index("s")
    col0 = c * 64
    row0 = s * (NP // 16)

    stage = pltpu.async_copy(
        h_hbm.at[pl.ds(row0, NP // 16), pl.ds(col0, 64)],
        h_sp.at[pl.ds(row0, NP // 16), :], g0_s)

    def zstore(i, carry):
        zed_v[i // 4, pl.ds((i % 4) * 16, 16)] = jnp.zeros((16,), jnp.float32)
        return carry

    lax.fori_loop(0, 64 * 4, zstore, 0)

    def zcp(i, carry):
        pltpu.sync_copy(zed_v, agg_sp.at[pl.ds(row0 + i * 64, 64), :])
        return carry

    lax.fori_loop(0, (NP // 16) // 64, zcp, 0)

    stage.wait()
    plsc.subcore_barrier()

    g0 = s * gpw

    def chunk(ci, carry):
        cg = g0 + ci * 16
        pltpu.sync_copy(sd_hbm.at[pl.ds(cg, 16), :], sd_v)
        src_j = lambda j: sd_v.at[j, pl.ds(0, 256)]
        dst_j = lambda j: sd_v.at[j, pl.ds(256, 256)]

        pltpu.async_copy(h_sp.at[src_j(0)], rows[0], gsem[0])

        def pair(jj, carry2):
            j0 = jj * 2
            j1 = j0 + 1
            pltpu.async_copy(h_sp.at[src_j(j1)], rows[1], gsem[1])
            pltpu.make_async_copy(h_sp.at[src_j(j0)], rows[0],
                                  gsem[0]).wait()
            pltpu.sync_copy(rows[0], agg_sp.at[dst_j(j0)], add=True)

            @pl.when(jj < 7)
            def _():
                pltpu.async_copy(h_sp.at[src_j(j0 + 2)], rows[0], gsem[0])

            pltpu.make_async_copy(h_sp.at[src_j(j1)], rows[1],
                                  gsem[1]).wait()
            pltpu.sync_copy(rows[1], agg_sp.at[dst_j(j1)], add=True)
            return carry2

        lax.fori_loop(0, 8, pair, 0)
        return carry

    lax.fori_loop(0, gpw // 16, chunk, 0)

    plsc.subcore_barrier()
    pltpu.sync_copy(agg_sp.at[pl.ds(row0, NP // 16), :],
                    agg_hbm.at[pl.ds(row0, NP // 16), pl.ds(col0, 64)])


@functools.lru_cache(maxsize=None)
def _segsum_call(gpw):
    return pl.kernel(
        functools.partial(_segsum_body, gpw),
        out_type=jax.ShapeDtypeStruct((NP, D), jnp.float32),
        mesh=_mesh(),
        compiler_params=pltpu.CompilerParams(use_tc_tiling_on_sc=False),
        scratch_types=[
            pltpu.VMEM((16, 512), jnp.int32),
            pltpu.VMEM((256, 64), jnp.float32),
            pltpu.VMEM((256, 64), jnp.float32),
            pltpu.VMEM((64, 64), jnp.float32),
            pltpu.VMEM_SHARED((NP, 64), jnp.float32),
            pltpu.VMEM_SHARED((NP, 64), jnp.float32),
            pltpu.SemaphoreType.DMA,
            pltpu.SemaphoreType.DMA,
        ],
    )


def _mlp_body(h_ref, agg_ref, w1_ref, b1_ref, w2_ref, b2_ref,
              ga_ref, be_ref, o_ref):
    z = h_ref[...] + agg_ref[...]
    z = jnp.maximum(
        jnp.dot(z, w1_ref[...], preferred_element_type=jnp.float32)
        + b1_ref[...], 0.0)
    z = jnp.maximum(
        jnp.dot(z, w2_ref[...], preferred_element_type=jnp.float32)
        + b2_ref[...], 0.0)
    msk = (lax.broadcasted_iota(jnp.int32, (NP, 1), 0) < N).astype(jnp.float32)
    zm = z * msk
    mean = jnp.sum(zm, axis=0, keepdims=True) * (1.0 / N)
    diff = (z - mean) * msk
    var = jnp.sum(diff * diff, axis=0, keepdims=True) * (1.0 / N)
    o_ref[...] = ((z - mean) * lax.rsqrt(var + 1e-5) * ga_ref[...]
                  + be_ref[...])


def _mlp(h, agg, w1, b1, w2, b2, ga, be):
    return pl.pallas_call(
        _mlp_body,
        out_shape=jax.ShapeDtypeStruct((NP, D), jnp.float32),
    )(h, agg, w1, b1, w2, b2, ga, be)


def _pool_body(z1_hbm, z2_hbm, z3_hbm, bt_hbm, out_hbm,
               bt_v, acc_v, bA1, bA2, bA3, bB1, bB2, bB3,
               sA1, sA2, sA3, sB1, sB2, sB3):
    set_a = ((bA1, sA1, z1_hbm), (bA2, sA2, z2_hbm), (bA3, sA3, z3_hbm))
    set_b = ((bB1, sB1, z1_hbm), (bB2, sB2, z2_hbm), (bB3, sB3, z3_hbm))
    c = lax.axis_index("c")
    s = lax.axis_index("s")
    t = s * 2 + c
    glo = t * 8
    ghi = glo + 8

    pltpu.sync_copy(bt_hbm, bt_v.at[pl.ds(0, NP)])
    bt_v[pl.ds(NP, 16)] = jnp.full((16,), 2 ** 30, jnp.int32)

    def lower_bound(val):
        def body(i, st):
            lo, hi = st
            mid = (lo + hi) // 2
            v = bt_v[pl.ds(mid, 16)][0]
            pred = v < val
            return (jnp.where(pred, mid + 1, lo), jnp.where(pred, hi, mid))

        return lax.fori_loop(0, 14, body, (jnp.int32(0), jnp.int32(NP)))[0]

    s0 = lower_bound(glo)
    e0 = lower_bound(ghi)

    neg = jnp.full((16,), -jnp.inf, dtype=jnp.float32)

    def ini(i, carry):
        acc_v[i // 24, pl.ds((i % 24) * 16, 16)] = neg
        return carry

    lax.fori_loop(0, 8 * 24, ini, 0)

    def issue(st, bi):
        r0 = pl.multiple_of(bi * 32, 32)
        for buf, sem, z in st:
            pltpu.async_copy(z.at[pl.ds(r0, 32), :], buf, sem)

    def wait(st, bi):
        r0 = pl.multiple_of(bi * 32, 32)
        for buf, sem, z in st:
            pltpu.make_async_copy(z.at[pl.ds(r0, 32), :], buf, sem).wait()

    def compute(st, bi):
        r0 = pl.multiple_of(bi * 32, 32)
        bufs = [x[0] for x in st]
        bv_lo = bt_v[pl.ds(r0, 16)]
        bv_hi = bt_v[pl.ds(r0 + 16, 16)]
        for k in range(32):
            r = r0 + k
            bv = bv_lo if k < 16 else bv_hi

            @pl.when((r >= s0) & (r < e0))
            def _(k=k, bv=bv):
                g = bv[k % 16] - glo
                for bidx, buf in enumerate(bufs):
                    def feat(j, c3, buf=buf, base=bidx * 128, k=k, g=g):
                        off = base + j * 16
                        acc_v[g, pl.ds(off, 16)] = jnp.maximum(
                            acc_v[g, pl.ds(off, 16)],
                            buf[k, pl.ds(j * 16, 16)])
                        return c3

                    lax.fori_loop(0, 8, feat, 0)

    b0 = s0 // 32
    bend = (e0 + 31) // 32

    @pl.when(bend > b0)
    def _():
        issue(set_a, b0)

    def blk2(jj, carry):
        i0 = b0 + 2 * jj
        i1 = i0 + 1

        @pl.when(i1 < bend)
        def _():
            issue(set_b, i1)

        wait(set_a, i0)
        compute(set_a, i0)

        @pl.when(i1 + 1 < bend)
        def _():
            issue(set_a, i0 + 2)

        @pl.when(i1 < bend)
        def _():
            wait(set_b, i1)
            compute(set_b, i1)

        return carry

    lax.fori_loop(0, (bend - b0 + 1) // 2, blk2, 0)

    pltpu.sync_copy(acc_v, out_hbm.at[pl.ds(glo, 8), :])


@functools.lru_cache(maxsize=None)
def _pool_call():
    return pl.kernel(
        _pool_body,
        out_type=jax.ShapeDtypeStruct((G, L * D), jnp.float32),
        mesh=_mesh(),
        scratch_types=[
            pltpu.VMEM((NP + 16,), jnp.int32),
            pltpu.VMEM((8, L * D), jnp.float32),
            pltpu.VMEM((32, D), jnp.float32),
            pltpu.VMEM((32, D), jnp.float32),
            pltpu.VMEM((32, D), jnp.float32),
            pltpu.VMEM((32, D), jnp.float32),
            pltpu.VMEM((32, D), jnp.float32),
            pltpu.VMEM((32, D), jnp.float32),
            pltpu.SemaphoreType.DMA,
            pltpu.SemaphoreType.DMA,
            pltpu.SemaphoreType.DMA,
            pltpu.SemaphoreType.DMA,
            pltpu.SemaphoreType.DMA,
            pltpu.SemaphoreType.DMA,
        ],
    )


def kernel(x, edge_index, batch, emb, W1, b1, W2, b2, gamma, beta):
    idx = x[:, 0].astype(jnp.int32)
    n = idx.shape[0]
    idx_p = jnp.pad(idx, (0, NP - n))
    bt_p = jnp.pad(batch.astype(jnp.int32), (0, NP - n), constant_values=G)

    src = edge_index[0].astype(jnp.int32)
    dst = edge_index[1].astype(jnp.int32)
    e = src.shape[0]
    egroups = -(-e // 256)
    gpw = -(-egroups // 16)
    gpw = -(-gpw // 16) * 16
    ep = gpw * 16 * 256
    src_p = jnp.pad(src, (0, ep - e)).reshape(gpw * 16, 256)
    dst_p = jnp.pad(dst, (0, ep - e), constant_values=NP - 1).reshape(
        gpw * 16, 256)
    sd_p = jnp.concatenate([src_p, dst_p], axis=1)

    h = _emb_call()(idx_p.reshape(NGROUPS, 128), emb)

    outs = []
    for i in range(L):
        agg = _segsum_call(gpw)(h, sd_p)
        h = _mlp(h, agg, W1[i], b1[i].reshape(1, D),
                 W2[i], b2[i].reshape(1, D),
                 gamma[i].reshape(1, D), beta[i].reshape(1, D))
        outs.append(h)

    return _pool_call()(outs[0], outs[1], outs[2], bt_p)
